# baseline (speedup 1.0000x reference)
import functools

import jax
import jax.numpy as jnp
from jax import lax
from jax.experimental import pallas as pl
from jax.experimental.pallas import tpu as pltpu
from jax.experimental.pallas import tpu_sc as plsc

N = 10000
E = 320000
D_IN = 128
H1 = 128
H2 = 64
K = 16

NC = 2
NS = 16
NW = NC * NS
VL = 16

DW = 128
LPT = DW // NW
NT = 10240
TRASH = N

CE = 512
EPAD = 512
NCHE = (E + EPAD) // CE
NBE = 2

CHUNK = 128
EPT_RAW = E // NW
NCH = 80
EPT = NCH * CHUNK
NPAD = NT
RPT = NPAD // NS

BNT = 1280

_MESH = plsc.VectorSubcoreMesh(core_axis_name="c", subcore_axis_name="s")


def _sc_aggregate():
    @functools.partial(
        pl.kernel,
        mesh=_MESH,
        out_type=jax.ShapeDtypeStruct((DW, NT), jnp.float32),
        compiler_params=pltpu.CompilerParams(needs_layout_passes=False),
        scratch_types=(
            [pltpu.VMEM((NT,), jnp.float32)] * LPT
            + [pltpu.VMEM((NT,), jnp.float32)] * LPT
            + [pltpu.VMEM((2 * CE,), jnp.int32)] * NBE
            + [pltpu.SemaphoreType.DMA] * NBE
        ),
    )
    def k(hpT_hbm, edges_hbm, out_hbm, *refs):
        hp_v = refs[:LPT]
        acc_v = refs[LPT:2 * LPT]
        ering = refs[2 * LPT:2 * LPT + NBE]
        sems = refs[2 * LPT + NBE:]

        cid = lax.axis_index("c")
        sid = lax.axis_index("s")
        wid = sid * NC + cid
        base = wid * LPT

        zer16 = jnp.zeros((VL,), jnp.float32)

        for l in range(LPT):
            pltpu.sync_copy(hpT_hbm.at[base + l], hp_v[l])

        @pl.loop(0, NT // VL)
        def _(i):
            for l in range(LPT):
                acc_v[l][pl.ds(i * VL, VL)] = zer16

        for b in range(NBE):
            pltpu.async_copy(edges_hbm.at[b], ering[b], sems[b])

        @pl.loop(0, NCHE, step=NBE)
        def _(c):
            for b in range(NBE):
                pltpu.make_async_copy(edges_hbm.at[0], ering[b],
                                      sems[b]).wait()
                for g in range(CE // VL):
                    s16 = ering[b][pl.ds(g * VL, VL)]
                    d16 = ering[b][pl.ds(CE + g * VL, VL)]
                    for l in range(LPT):
                        v = plsc.load_gather(hp_v[l], [s16])
                        plsc.addupdate_scatter(acc_v[l], [d16], v)

                @pl.when(c + b + NBE < NCHE)
                def _():
                    pltpu.async_copy(edges_hbm.at[c + b + NBE], ering[b],
                                     sems[b])

        for l in range(LPT):
            pltpu.sync_copy(acc_v[l], out_hbm.at[base + l])

    return k


def _sc_degree():
    @functools.partial(
        pl.kernel,
        mesh=_MESH,
        out_type=jax.ShapeDtypeStruct((NC, NPAD, DW), jnp.float32),
        scratch_types=[
            pltpu.VMEM((NCH, CHUNK), jnp.int32),
            pltpu.VMEM((CHUNK, DW), jnp.float32),
            pltpu.VMEM((CHUNK, DW), jnp.float32),
            pltpu.VMEM_SHARED((NPAD, DW), jnp.float32),
        ],
    )
    def k(dst_hbm, zeros_hbm, ones_hbm, out_hbm, dst_v, zrows_v, ones_v, acc_sh):
        cid = lax.axis_index("c")
        sid = lax.axis_index("s")
        wid = sid * NC + cid

        pltpu.sync_copy(zeros_hbm, zrows_v)
        pltpu.sync_copy(ones_hbm, ones_v)

        @pl.loop(0, RPT, step=CHUNK)
        def _(r):
            pltpu.sync_copy(zrows_v, acc_sh.at[pl.ds(sid * RPT + r, CHUNK)])

        pltpu.sync_copy(dst_hbm.at[wid], dst_v)
        plsc.subcore_barrier()

        @pl.loop(0, NCH)
        def _(j):
            pltpu.sync_copy(ones_v, acc_sh.at[dst_v.at[j]], add=True)

        plsc.subcore_barrier()

        @pl.loop(0, RPT, step=CHUNK)
        def _(r):
            pltpu.sync_copy(acc_sh.at[pl.ds(sid * RPT + r, CHUNK)],
                            out_hbm.at[cid].at[pl.ds(sid * RPT + r, CHUNK)])

    return k


def _dotT(a, b):
    return lax.dot_general(a, b, (((0,), (0,)), ((), ())),
                           precision=lax.Precision.HIGHEST,
                           preferred_element_type=jnp.float32)


def _tc_dinv(dp):
    def body(dp_ref, out_ref):
        d = dp_ref[...]
        out_ref[...] = lax.rsqrt(d[0, :, 0:1] + d[1, :, 0:1] + 1.0)

    return pl.pallas_call(
        body,
        grid=(NT // BNT,),
        in_specs=[pl.BlockSpec((NC, BNT, DW), lambda i: (0, i, 0))],
        out_specs=pl.BlockSpec((BNT, 1), lambda i: (i, 0)),
        out_shape=jax.ShapeDtypeStruct((NPAD, 1), jnp.float32),
    )(dp)


def _tc_stage1(xT, w1, dinvT):
    def body(x_ref, w_ref, dinv_ref, hp_ref):
        g = _dotT(w_ref[...], x_ref[...])
        hp_ref[...] = dinv_ref[...] * g

    return pl.pallas_call(
        body,
        grid=(NT // BNT,),
        in_specs=[
            pl.BlockSpec((D_IN, BNT), lambda i: (0, i)),
            pl.BlockSpec((D_IN, H1), lambda i: (0, 0)),
            pl.BlockSpec((1, BNT), lambda i: (0, i)),
        ],
        out_specs=pl.BlockSpec((H1, BNT), lambda i: (0, i)),
        out_shape=jax.ShapeDtypeStruct((H1, NT), jnp.float32),
    )(xT, w1, dinvT)


def _tc_stage_mid(accT, hpT, dinvT, b_col, wn):
    def body(acc_ref, hp_ref, dinv_ref, b_ref, w_ref, out_ref):
        dinv = dinv_ref[...]
        s = acc_ref[...] + hp_ref[...]
        h = jnp.maximum(dinv * s + b_ref[...], 0.0)
        out_ref[...] = dinv * _dotT(w_ref[...], h)

    return pl.pallas_call(
        body,
        grid=(NT // BNT,),
        in_specs=[
            pl.BlockSpec((DW, BNT), lambda i: (0, i)),
            pl.BlockSpec((DW, BNT), lambda i: (0, i)),
            pl.BlockSpec((1, BNT), lambda i: (0, i)),
            pl.BlockSpec((DW, 1), lambda i: (0, 0)),
            pl.BlockSpec((DW, DW), lambda i: (0, 0)),
        ],
        out_specs=pl.BlockSpec((DW, BNT), lambda i: (0, i)),
        out_shape=jax.ShapeDtypeStruct((DW, NT), jnp.float32),
    )(accT, hpT, dinvT, b_col, wn)


def _tc_stage3(accT, hpT, dinvT, b_col):
    def body(acc_ref, hp_ref, dinv_ref, b_ref, out_ref):
        full = dinv_ref[...] * (acc_ref[...] + hp_ref[...])
        logits = full[0:K, :] + b_ref[...]
        m = jnp.max(logits, axis=0, keepdims=True)
        e = jnp.exp(logits - m)
        out_ref[...] = e / jnp.sum(e, axis=0, keepdims=True)

    return pl.pallas_call(
        body,
        grid=(NT // BNT,),
        in_specs=[
            pl.BlockSpec((DW, BNT), lambda i: (0, i)),
            pl.BlockSpec((DW, BNT), lambda i: (0, i)),
            pl.BlockSpec((1, BNT), lambda i: (0, i)),
            pl.BlockSpec((K, 1), lambda i: (0, 0)),
        ],
        out_specs=pl.BlockSpec((K, BNT), lambda i: (0, i)),
        out_shape=jax.ShapeDtypeStruct((K, NT), jnp.float32),
    )(accT, hpT, dinvT, b_col)


def kernel(x, edge_index, W1, b1, W2, b2, W3, b3):
    xT = jnp.pad(x.T, ((0, 0), (0, NT - N)))

    src = jnp.pad(edge_index[0], (0, EPAD)).reshape(NCHE, CE)
    dst = jnp.pad(edge_index[1], (0, EPAD),
                  constant_values=TRASH).reshape(NCHE, CE)
    echunks = jnp.stack([src, dst], axis=1).reshape(NCHE, 2 * CE)

    dsth = edge_index[1].reshape(NW, EPT_RAW)
    dsth = jnp.pad(dsth, ((0, 0), (0, EPT - EPT_RAW)),
                   constant_values=TRASH).reshape(NW, NCH, CHUNK)
    zdw = jnp.zeros((CHUNK, DW), jnp.float32)
    onesdw = jnp.ones((CHUNK, DW), jnp.float32)

    w2p = jnp.zeros((DW, DW), jnp.float32).at[:H1, :H2].set(W2)
    w3p = jnp.zeros((DW, DW), jnp.float32).at[:H2, :K].set(W3)
    b1c = b1.reshape(H1, 1)
    b2c = jnp.zeros((DW, 1), jnp.float32).at[:H2, 0].set(b2)
    b3c = b3.reshape(K, 1)

    agg = _sc_aggregate()

    dp = _sc_degree()(dsth, zdw, onesdw)
    dinv = _tc_dinv(dp).reshape(1, NT)

    hp1 = _tc_stage1(xT, W1, dinv)
    acc1 = agg(hp1, echunks)
    hp2 = _tc_stage_mid(acc1, hp1, dinv, b1c, w2p)
    acc2 = agg(hp2, echunks)
    hp3 = _tc_stage_mid(acc2, hp2, dinv, b2c, w3p)
    acc3 = agg(hp3, echunks)

    outT = _tc_stage3(acc3, hp3, dinv, b3c)
    return outT[:, :N].T


# speedup vs baseline: 1.2368x; 1.2368x over previous
import functools

import jax
import jax.numpy as jnp
from jax import lax
from jax.experimental import pallas as pl
from jax.experimental.pallas import tpu as pltpu
from jax.experimental.pallas import tpu_sc as plsc

N = 10000
E = 320000
D_IN = 128
H1 = 128
H2 = 64
K = 16

NC = 2
NS = 16
NW = NC * NS
LANES = 16

CHUNK = 128
EPT_RAW = E // NW
NCH = 80
EPT = NCH * CHUNK
NPAD = 10240
RPT = NPAD // NS
TRASH = N

DW = 128
BR = 1000

_MESH = plsc.VectorSubcoreMesh(core_axis_name="c", subcore_axis_name="s")


NBUF = 2
NSEG = 2
NCHS = NCH // NSEG
HALF = CHUNK // 2


def _sc_aggregate():
    d = DW

    @functools.partial(
        pl.kernel,
        mesh=_MESH,
        out_type=jax.ShapeDtypeStruct((NC, NPAD, d), jnp.float32),
        scratch_types=[
            pltpu.VMEM((NCHS, CHUNK), jnp.int32),
            pltpu.VMEM((NCHS, CHUNK), jnp.int32),
            pltpu.VMEM((NBUF, CHUNK, d), jnp.float32),
            pltpu.VMEM_SHARED((NPAD, d), jnp.float32),
        ] + [pltpu.SemaphoreType.DMA] * NBUF,
    )
    def k(hp_hbm, src_hbm, dst_hbm, zeros_hbm, out_hbm,
          src_v, dst_v, rows_v, acc_sh, *sems):
        cid = lax.axis_index("c")
        sid = lax.axis_index("s")
        wid = sid * NC + cid

        pltpu.sync_copy(zeros_hbm, rows_v.at[0])

        @pl.loop(0, RPT, step=CHUNK)
        def _(r):
            pltpu.sync_copy(rows_v.at[0], acc_sh.at[pl.ds(sid * RPT + r, CHUNK)])

        plsc.subcore_barrier()

        @pl.loop(0, NSEG)
        def _(seg):
            pltpu.sync_copy(src_hbm.at[wid].at[seg], src_v)
            pltpu.sync_copy(dst_hbm.at[wid].at[seg], dst_v)

            def issue(c, b):
                for h in (0, HALF):
                    pltpu.async_copy(
                        hp_hbm.at[src_v.at[c].at[pl.ds(h, HALF)]],
                        rows_v.at[b].at[pl.ds(h, HALF)], sems[b])

            for b in range(NBUF):
                issue(b, b)

            @pl.loop(0, NCHS, step=NBUF)
            def _(j):
                for b in range(NBUF):
                    pltpu.make_async_copy(zeros_hbm, rows_v.at[b],
                                          sems[b]).wait()
                    pltpu.sync_copy(rows_v.at[b], acc_sh.at[dst_v.at[j + b]],
                                    add=True)

                    @pl.when(j + b + NBUF < NCHS)
                    def _():
                        issue(j + b + NBUF, b)

        plsc.subcore_barrier()

        @pl.loop(0, RPT, step=CHUNK)
        def _(r):
            pltpu.sync_copy(acc_sh.at[pl.ds(sid * RPT + r, CHUNK)],
                            out_hbm.at[cid].at[pl.ds(sid * RPT + r, CHUNK)])

    return k


def _sc_degree():
    @functools.partial(
        pl.kernel,
        mesh=_MESH,
        out_type=jax.ShapeDtypeStruct((NC, NPAD, DW), jnp.float32),
        scratch_types=[
            pltpu.VMEM((NCH, CHUNK), jnp.int32),
            pltpu.VMEM((CHUNK, DW), jnp.float32),
            pltpu.VMEM((CHUNK, DW), jnp.float32),
            pltpu.VMEM_SHARED((NPAD, DW), jnp.float32),
        ],
    )
    def k(dst_hbm, zeros_hbm, ones_hbm, out_hbm, dst_v, zrows_v, ones_v, acc_sh):
        cid = lax.axis_index("c")
        sid = lax.axis_index("s")
        wid = sid * NC + cid

        pltpu.sync_copy(zeros_hbm, zrows_v)
        pltpu.sync_copy(ones_hbm, ones_v)

        @pl.loop(0, RPT, step=CHUNK)
        def _(r):
            pltpu.sync_copy(zrows_v, acc_sh.at[pl.ds(sid * RPT + r, CHUNK)])

        pltpu.sync_copy(dst_hbm.at[wid], dst_v)
        plsc.subcore_barrier()

        @pl.loop(0, NCH)
        def _(j):
            pltpu.sync_copy(ones_v, acc_sh.at[dst_v.at[j]], add=True)

        plsc.subcore_barrier()

        @pl.loop(0, RPT, step=CHUNK)
        def _(r):
            pltpu.sync_copy(acc_sh.at[pl.ds(sid * RPT + r, CHUNK)],
                            out_hbm.at[cid].at[pl.ds(sid * RPT + r, CHUNK)])

    return k


def _dot(a, b):
    return jnp.dot(a, b, precision=lax.Precision.HIGHEST,
                   preferred_element_type=jnp.float32)


def _tc_stage1(x, w1, dp):
    def body(x_ref, w_ref, dp_ref, hp_ref, dinv_ref):
        dp = dp_ref[...]
        deg = dp[0, :, 0:1] + dp[1, :, 0:1] + 1.0
        dinv = lax.rsqrt(deg)
        g = _dot(x_ref[...], w_ref[...])
        hp_ref[...] = dinv * g
        dinv_ref[...] = dinv

    return pl.pallas_call(
        body,
        grid=(N // BR,),
        in_specs=[
            pl.BlockSpec((BR, D_IN), lambda i: (i, 0)),
            pl.BlockSpec((D_IN, H1), lambda i: (0, 0)),
            pl.BlockSpec((NC, BR, DW), lambda i: (0, i, 0)),
        ],
        out_specs=[
            pl.BlockSpec((BR, H1), lambda i: (i, 0)),
            pl.BlockSpec((BR, 1), lambda i: (i, 0)),
        ],
        out_shape=[
            jax.ShapeDtypeStruct((N, H1), jnp.float32),
            jax.ShapeDtypeStruct((N, 1), jnp.float32),
        ],
    )(x, w1, dp)


def _tc_stage_mid(acc, hp, dinv, b, wn):
    def body(acc_ref, hp_ref, dinv_ref, b_ref, w_ref, out_ref):
        a = acc_ref[...]
        dinv = dinv_ref[...]
        s = a[0] + a[1] + hp_ref[...]
        h = jnp.maximum(dinv * s + b_ref[...], 0.0)
        out_ref[...] = dinv * _dot(h, w_ref[...])

    return pl.pallas_call(
        body,
        grid=(N // BR,),
        in_specs=[
            pl.BlockSpec((NC, BR, DW), lambda i: (0, i, 0)),
            pl.BlockSpec((BR, DW), lambda i: (i, 0)),
            pl.BlockSpec((BR, 1), lambda i: (i, 0)),
            pl.BlockSpec((1, DW), lambda i: (0, 0)),
            pl.BlockSpec((DW, DW), lambda i: (0, 0)),
        ],
        out_specs=pl.BlockSpec((BR, DW), lambda i: (i, 0)),
        out_shape=jax.ShapeDtypeStruct((N, DW), jnp.float32),
    )(acc, hp, dinv, b, wn)


def _tc_stage3(acc, hp, dinv, b):
    def body(acc_ref, hp_ref, dinv_ref, b_ref, out_ref):
        a = acc_ref[...]
        full = dinv_ref[...] * (a[0] + a[1] + hp_ref[...])
        logits = full[:, 0:K] + b_ref[...]
        m = jnp.max(logits, axis=1, keepdims=True)
        e = jnp.exp(logits - m)
        out_ref[...] = e / jnp.sum(e, axis=1, keepdims=True)

    return pl.pallas_call(
        body,
        grid=(N // BR,),
        in_specs=[
            pl.BlockSpec((NC, BR, DW), lambda i: (0, i, 0)),
            pl.BlockSpec((BR, DW), lambda i: (i, 0)),
            pl.BlockSpec((BR, 1), lambda i: (i, 0)),
            pl.BlockSpec((1, K), lambda i: (0, 0)),
        ],
        out_specs=pl.BlockSpec((BR, K), lambda i: (i, 0)),
        out_shape=jax.ShapeDtypeStruct((N, K), jnp.float32),
    )(acc, hp, dinv, b)


def kernel(x, edge_index, W1, b1, W2, b2, W3, b3):
    src = edge_index[0].reshape(NW, EPT_RAW)
    dst = edge_index[1].reshape(NW, EPT_RAW)
    pad = EPT - EPT_RAW
    src = jnp.pad(src, ((0, 0), (0, pad))).reshape(NW, NSEG, NCHS, CHUNK)
    dst = jnp.pad(dst, ((0, 0), (0, pad)),
                  constant_values=TRASH).reshape(NW, NSEG, NCHS, CHUNK)

    zdw = jnp.zeros((CHUNK, DW), jnp.float32)
    onesdw = jnp.ones((CHUNK, DW), jnp.float32)

    w2p = jnp.zeros((DW, DW), jnp.float32).at[:H1, :H2].set(W2)
    w3p = jnp.zeros((DW, DW), jnp.float32).at[:H2, :K].set(W3)
    b2p = jnp.zeros((1, DW), jnp.float32).at[0, :H2].set(b2)

    agg = _sc_aggregate()

    dp = _sc_degree()(dst.reshape(NW, NCH, CHUNK), zdw, onesdw)

    hp1, dinv = _tc_stage1(x, W1, dp)
    acc1 = agg(hp1, src, dst, zdw)
    hp2 = _tc_stage_mid(acc1, hp1, dinv, b1.reshape(1, H1), w2p)
    acc2 = agg(hp2, src, dst, zdw)
    hp3 = _tc_stage_mid(acc2, hp2, dinv, b2p, w3p)
    acc3 = agg(hp3, src, dst, zdw)

    return _tc_stage3(acc3, hp3, dinv, b3.reshape(1, K))


# speedup vs baseline: 1.2388x; 1.0017x over previous
import functools

import jax
import jax.numpy as jnp
from jax import lax
from jax.experimental import pallas as pl
from jax.experimental.pallas import tpu as pltpu
from jax.experimental.pallas import tpu_sc as plsc

N = 10000
E = 320000
D_IN = 128
H1 = 128
H2 = 64
K = 16

NC = 2
NS = 16
NW = NC * NS
LANES = 16

CHUNK = 128
EPT_RAW = E // NW
NCH = 80
EPT = NCH * CHUNK
NPAD = 10240
RPT = NPAD // NS
TRASH = N

DW = 128
BR = 1000

_MESH = plsc.VectorSubcoreMesh(core_axis_name="c", subcore_axis_name="s")


NBUF = 2
NSEG = 2
NCHS = NCH // NSEG
HALF = CHUNK // 2
QTR = CHUNK // 4


def _sc_aggregate():
    d = DW

    @functools.partial(
        pl.kernel,
        mesh=_MESH,
        out_type=jax.ShapeDtypeStruct((NC, NPAD, d), jnp.float32),
        scratch_types=[
            pltpu.VMEM((NCHS, CHUNK), jnp.int32),
            pltpu.VMEM((NCHS, CHUNK), jnp.int32),
            pltpu.VMEM((NBUF, CHUNK, d), jnp.float32),
            pltpu.VMEM_SHARED((NPAD, d), jnp.float32),
        ] + [pltpu.SemaphoreType.DMA] * NBUF,
    )
    def k(hp_hbm, src_hbm, dst_hbm, zeros_hbm, out_hbm,
          src_v, dst_v, rows_v, acc_sh, *sems):
        cid = lax.axis_index("c")
        sid = lax.axis_index("s")
        wid = sid * NC + cid

        pltpu.sync_copy(zeros_hbm, rows_v.at[0])

        @pl.loop(0, RPT, step=CHUNK)
        def _(r):
            pltpu.sync_copy(rows_v.at[0], acc_sh.at[pl.ds(sid * RPT + r, CHUNK)])

        plsc.subcore_barrier()

        @pl.loop(0, NSEG)
        def _(seg):
            pltpu.sync_copy(src_hbm.at[wid].at[seg], src_v)
            pltpu.sync_copy(dst_hbm.at[wid].at[seg], dst_v)

            def issue(c, b):
                for h in (0, QTR, 2 * QTR, 3 * QTR):
                    pltpu.async_copy(
                        hp_hbm.at[src_v.at[c].at[pl.ds(h, QTR)]],
                        rows_v.at[b].at[pl.ds(h, QTR)], sems[b])

            for b in range(NBUF):
                issue(b, b)

            @pl.loop(0, NCHS, step=NBUF)
            def _(j):
                for b in range(NBUF):
                    pltpu.make_async_copy(zeros_hbm, rows_v.at[b],
                                          sems[b]).wait()
                    pltpu.sync_copy(rows_v.at[b], acc_sh.at[dst_v.at[j + b]],
                                    add=True)

                    @pl.when(j + b + NBUF < NCHS)
                    def _():
                        issue(j + b + NBUF, b)

        plsc.subcore_barrier()

        @pl.loop(0, RPT, step=CHUNK)
        def _(r):
            pltpu.sync_copy(acc_sh.at[pl.ds(sid * RPT + r, CHUNK)],
                            out_hbm.at[cid].at[pl.ds(sid * RPT + r, CHUNK)])

    return k


def _sc_degree():
    @functools.partial(
        pl.kernel,
        mesh=_MESH,
        out_type=jax.ShapeDtypeStruct((NC, NPAD, DW), jnp.float32),
        scratch_types=[
            pltpu.VMEM((NCH, CHUNK), jnp.int32),
            pltpu.VMEM((CHUNK, DW), jnp.float32),
            pltpu.VMEM((CHUNK, DW), jnp.float32),
            pltpu.VMEM_SHARED((NPAD, DW), jnp.float32),
        ],
    )
    def k(dst_hbm, zeros_hbm, ones_hbm, out_hbm, dst_v, zrows_v, ones_v, acc_sh):
        cid = lax.axis_index("c")
        sid = lax.axis_index("s")
        wid = sid * NC + cid

        pltpu.sync_copy(zeros_hbm, zrows_v)
        pltpu.sync_copy(ones_hbm, ones_v)

        @pl.loop(0, RPT, step=CHUNK)
        def _(r):
            pltpu.sync_copy(zrows_v, acc_sh.at[pl.ds(sid * RPT + r, CHUNK)])

        pltpu.sync_copy(dst_hbm.at[wid], dst_v)
        plsc.subcore_barrier()

        @pl.loop(0, NCH)
        def _(j):
            pltpu.sync_copy(ones_v, acc_sh.at[dst_v.at[j]], add=True)

        plsc.subcore_barrier()

        @pl.loop(0, RPT, step=CHUNK)
        def _(r):
            pltpu.sync_copy(acc_sh.at[pl.ds(sid * RPT + r, CHUNK)],
                            out_hbm.at[cid].at[pl.ds(sid * RPT + r, CHUNK)])

    return k


def _dot(a, b):
    return jnp.dot(a, b, precision=lax.Precision.HIGHEST,
                   preferred_element_type=jnp.float32)


def _tc_stage1(x, w1, dp):
    def body(x_ref, w_ref, dp_ref, hp_ref, dinv_ref):
        dp = dp_ref[...]
        deg = dp[0, :, 0:1] + dp[1, :, 0:1] + 1.0
        dinv = lax.rsqrt(deg)
        g = _dot(x_ref[...], w_ref[...])
        hp_ref[...] = dinv * g
        dinv_ref[...] = dinv

    return pl.pallas_call(
        body,
        grid=(N // BR,),
        in_specs=[
            pl.BlockSpec((BR, D_IN), lambda i: (i, 0)),
            pl.BlockSpec((D_IN, H1), lambda i: (0, 0)),
            pl.BlockSpec((NC, BR, DW), lambda i: (0, i, 0)),
        ],
        out_specs=[
            pl.BlockSpec((BR, H1), lambda i: (i, 0)),
            pl.BlockSpec((BR, 1), lambda i: (i, 0)),
        ],
        out_shape=[
            jax.ShapeDtypeStruct((N, H1), jnp.float32),
            jax.ShapeDtypeStruct((N, 1), jnp.float32),
        ],
    )(x, w1, dp)


def _tc_stage_mid(acc, hp, dinv, b, wn):
    def body(acc_ref, hp_ref, dinv_ref, b_ref, w_ref, out_ref):
        a = acc_ref[...]
        dinv = dinv_ref[...]
        s = a[0] + a[1] + hp_ref[...]
        h = jnp.maximum(dinv * s + b_ref[...], 0.0)
        out_ref[...] = dinv * _dot(h, w_ref[...])

    return pl.pallas_call(
        body,
        grid=(N // BR,),
        in_specs=[
            pl.BlockSpec((NC, BR, DW), lambda i: (0, i, 0)),
            pl.BlockSpec((BR, DW), lambda i: (i, 0)),
            pl.BlockSpec((BR, 1), lambda i: (i, 0)),
            pl.BlockSpec((1, DW), lambda i: (0, 0)),
            pl.BlockSpec((DW, DW), lambda i: (0, 0)),
        ],
        out_specs=pl.BlockSpec((BR, DW), lambda i: (i, 0)),
        out_shape=jax.ShapeDtypeStruct((N, DW), jnp.float32),
    )(acc, hp, dinv, b, wn)


def _tc_stage3(acc, hp, dinv, b):
    def body(acc_ref, hp_ref, dinv_ref, b_ref, out_ref):
        a = acc_ref[...]
        full = dinv_ref[...] * (a[0] + a[1] + hp_ref[...])
        logits = full[:, 0:K] + b_ref[...]
        m = jnp.max(logits, axis=1, keepdims=True)
        e = jnp.exp(logits - m)
        out_ref[...] = e / jnp.sum(e, axis=1, keepdims=True)

    return pl.pallas_call(
        body,
        grid=(N // BR,),
        in_specs=[
            pl.BlockSpec((NC, BR, DW), lambda i: (0, i, 0)),
            pl.BlockSpec((BR, DW), lambda i: (i, 0)),
            pl.BlockSpec((BR, 1), lambda i: (i, 0)),
            pl.BlockSpec((1, K), lambda i: (0, 0)),
        ],
        out_specs=pl.BlockSpec((BR, K), lambda i: (i, 0)),
        out_shape=jax.ShapeDtypeStruct((N, K), jnp.float32),
    )(acc, hp, dinv, b)


def kernel(x, edge_index, W1, b1, W2, b2, W3, b3):
    src = edge_index[0].reshape(NW, EPT_RAW)
    dst = edge_index[1].reshape(NW, EPT_RAW)
    pad = EPT - EPT_RAW
    src = jnp.pad(src, ((0, 0), (0, pad))).reshape(NW, NSEG, NCHS, CHUNK)
    dst = jnp.pad(dst, ((0, 0), (0, pad)),
                  constant_values=TRASH).reshape(NW, NSEG, NCHS, CHUNK)

    zdw = jnp.zeros((CHUNK, DW), jnp.float32)
    onesdw = jnp.ones((CHUNK, DW), jnp.float32)

    w2p = jnp.zeros((DW, DW), jnp.float32).at[:H1, :H2].set(W2)
    w3p = jnp.zeros((DW, DW), jnp.float32).at[:H2, :K].set(W3)
    b2p = jnp.zeros((1, DW), jnp.float32).at[0, :H2].set(b2)

    agg = _sc_aggregate()

    dp = _sc_degree()(dst.reshape(NW, NCH, CHUNK), zdw, onesdw)

    hp1, dinv = _tc_stage1(x, W1, dp)
    acc1 = agg(hp1, src, dst, zdw)
    hp2 = _tc_stage_mid(acc1, hp1, dinv, b1.reshape(1, H1), w2p)
    acc2 = agg(hp2, src, dst, zdw)
    hp3 = _tc_stage_mid(acc2, hp2, dinv, b2p, w3p)
    acc3 = agg(hp3, src, dst, zdw)

    return _tc_stage3(acc3, hp3, dinv, b3.reshape(1, K))
